# Initial kernel scaffold; baseline (speedup 1.0000x reference)
import jax
import jax.numpy as jnp
from jax.experimental import pallas as pl


def kernel(z, edge_index):
    raise NotImplementedError("write your pallas kernel here")


# speedup vs baseline: 1.7823x; 1.7823x over previous
import jax
import jax.numpy as jnp
from jax import lax
from jax.experimental import pallas as pl
from jax.experimental.pallas import tpu as pltpu
from jax.experimental.pallas import tpu_sc as plsc

NC = 2
NS = 16
L = 16
NW = NC * NS

E = 160000
D = 256
B = 64
T0 = 118
T1 = 42
EP = NS * (T0 + T1) * B
TS = 65
PMAX = max(T0, T1) * B


def _body(z_hbm, src_hbm, dst_hbm, out_hbm, idx_s, idx_d, rows_s, rows_d,
          tmat, out_loc, sem_s0, sem_d0, sem_s1, sem_d1):
    c = lax.axis_index("c")
    s = lax.axis_index("s")
    is0 = c == 0
    t_w = jnp.where(is0, T0, T1)
    base0 = jnp.where(is0, s * (T0 * B), NS * (T0 * B) + s * (T1 * B))
    sems = ((sem_s0, sem_d0), (sem_s1, sem_d1))

    @pl.when(is0)
    def _():
        pltpu.sync_copy(src_hbm.at[pl.ds(base0, T0 * B)],
                        idx_s.at[pl.ds(0, T0 * B)])
        pltpu.sync_copy(dst_hbm.at[pl.ds(base0, T0 * B)],
                        idx_d.at[pl.ds(0, T0 * B)])

    @pl.when(jnp.logical_not(is0))
    def _():
        pltpu.sync_copy(src_hbm.at[pl.ds(base0, T1 * B)],
                        idx_s.at[pl.ds(0, T1 * B)])
        pltpu.sync_copy(dst_hbm.at[pl.ds(base0, T1 * B)],
                        idx_d.at[pl.ds(0, T1 * B)])

    def start(it, p):
        pltpu.async_copy(z_hbm.at[idx_s.at[pl.ds(it * B, B)]],
                         rows_s.at[p], sems[p][0])
        pltpu.async_copy(z_hbm.at[idx_d.at[pl.ds(it * B, B)]],
                         rows_d.at[p], sems[p][1])

    def wait(p):
        pltpu.make_async_copy(z_hbm.at[idx_s.at[pl.ds(0, B)]],
                              rows_s.at[p], sems[p][0]).wait()
        pltpu.make_async_copy(z_hbm.at[idx_d.at[pl.ds(0, B)]],
                              rows_d.at[p], sems[p][1]).wait()

    start(0, 0)
    start(1, 1)

    lane = lax.iota(jnp.int32, L)
    col0 = lane * TS

    @pl.loop(0, t_w, step=2)
    def _iter(i):
        for p in range(2):
            it = i + p
            wait(p)
            rs, rd = rows_s.at[p], rows_d.at[p]

            @pl.loop(0, B, init_carry=col0, unroll=4)
            def _edge(e, idxv):
                parts = []
                for j in range(4):
                    q = rs[e, pl.ds(j * 64, L)] * rd[e, pl.ds(j * 64, L)]
                    for k in range(1, 4):
                        off = j * 64 + k * L
                        q = q + rs[e, pl.ds(off, L)] * rd[e, pl.ds(off, L)]
                    parts.append(q)
                acc = (parts[0] + parts[1]) + (parts[2] + parts[3])
                plsc.store_scatter(tmat, [idxv], acc)
                return idxv + 1

            for g in range(B // L):
                ssum = tmat[pl.ds(g * L, L)]
                for l in range(1, L):
                    ssum = ssum + tmat[pl.ds(l * TS + g * L, L)]
                sig = 1.0 / (1.0 + jnp.exp(-ssum))
                out_loc[pl.ds(it * B + g * L, L)] = sig

            @pl.when(it + 2 < t_w)
            def _():
                start(it + 2, p)

    @pl.when(is0)
    def _():
        pltpu.sync_copy(out_loc.at[pl.ds(0, T0 * B)],
                        out_hbm.at[pl.ds(base0, T0 * B)])

    @pl.when(jnp.logical_not(is0))
    def _():
        pltpu.sync_copy(out_loc.at[pl.ds(0, T1 * B)],
                        out_hbm.at[pl.ds(base0, T1 * B)])


@jax.jit
def _gae_decode(z, src, dst):
    mesh = plsc.VectorSubcoreMesh(core_axis_name="c", subcore_axis_name="s",
                                  num_cores=NC, num_subcores=NS)
    return pl.kernel(
        _body,
        out_type=jax.ShapeDtypeStruct((EP,), jnp.float32),
        mesh=mesh,
        compiler_params=pltpu.CompilerParams(needs_layout_passes=False),
        scratch_types=[
            pltpu.VMEM((PMAX,), jnp.int32),
            pltpu.VMEM((PMAX,), jnp.int32),
            pltpu.VMEM((2, B, D), jnp.float32),
            pltpu.VMEM((2, B, D), jnp.float32),
            pltpu.VMEM(((L - 1) * TS + B, ), jnp.float32),
            pltpu.VMEM((PMAX,), jnp.float32),
            pltpu.SemaphoreType.DMA,
            pltpu.SemaphoreType.DMA,
            pltpu.SemaphoreType.DMA,
            pltpu.SemaphoreType.DMA,
        ],
    )(z, src, dst)


def kernel(z, edge_index):
    src = jnp.pad(edge_index[0].astype(jnp.int32), (0, EP - E))
    dst = jnp.pad(edge_index[1].astype(jnp.int32), (0, EP - E))
    return _gae_decode(z, src, dst)[:E]
